# baseline (speedup 1.0000x reference)
import jax
import jax.numpy as jnp
from jax.experimental import pallas as pl


_S_BLK = 256


def _pe_add_kernel(x_ref, w_ref, o_ref):
    w = w_ref[...]
    for b in range(x_ref.shape[1]):
        o_ref[:, b, :] = x_ref[:, b, :] + w


def kernel(x, pos_embed_weight):
    seq_len, batch, d_model = x.shape
    grid = (seq_len // _S_BLK,)
    return pl.pallas_call(
        _pe_add_kernel,
        grid=grid,
        in_specs=[
            pl.BlockSpec((_S_BLK, batch, d_model), lambda i: (i, 0, 0)),
            pl.BlockSpec((_S_BLK, d_model), lambda i: (i, 0)),
        ],
        out_specs=pl.BlockSpec((_S_BLK, batch, d_model), lambda i: (i, 0, 0)),
        out_shape=jax.ShapeDtypeStruct(x.shape, x.dtype),
    )(x, pos_embed_weight)
